# baseline (speedup 1.0000x reference)
import functools

import jax
import jax.numpy as jnp
from jax import lax
from jax.experimental import pallas as pl
from jax.experimental.pallas import tpu as pltpu
from jax.experimental.pallas import tpu_sc as plsc

_V = 100000
_D = 64
_B = 1024
_S = 200
_NSEG = 3
_NTOK = _B * _S

_NC = 2
_NS = 16
_NW = _NC * _NS
_TPW = _NTOK // _NW
_CHUNK = 640
_NCHUNK = _TPW // _CHUNK
_GSUB = 128


@functools.cache
def _make_sc_gather():
    mesh = plsc.VectorSubcoreMesh(core_axis_name="c", subcore_axis_name="s")

    @functools.partial(
        pl.kernel,
        mesh=mesh,
        out_type=jax.ShapeDtypeStruct((_NTOK, _D), jnp.float32),
        scratch_types=[
            pltpu.VMEM((_CHUNK,), jnp.int32),
            pltpu.VMEM((_CHUNK, _D), jnp.float32),
            pltpu.SemaphoreType.DMA,
        ],
        compiler_params=pltpu.CompilerParams(use_tc_tiling_on_sc=False),
    )
    def gather_k(idx_hbm, table_hbm, out_hbm, idx_v, rows_v, sem):
        wid = lax.axis_index("s") * _NC + lax.axis_index("c")
        base0 = wid * _TPW
        for c in range(_NCHUNK):
            base = base0 + c * _CHUNK
            pltpu.sync_copy(idx_hbm.at[pl.ds(base, _CHUNK)], idx_v)
            copies = []
            for j in range(_CHUNK // _GSUB):
                copies.append(
                    pltpu.async_copy(
                        table_hbm.at[idx_v.at[pl.ds(j * _GSUB, _GSUB)]],
                        rows_v.at[pl.ds(j * _GSUB, _GSUB)],
                        sem,
                    )
                )
            for cp in copies:
                cp.wait()
            pltpu.sync_copy(rows_v, out_hbm.at[pl.ds(base, _CHUNK)])

    return gather_k


@functools.cache
def _make_sc_fused():
    mesh = plsc.VectorSubcoreMesh(core_axis_name="c", subcore_axis_name="s")

    @functools.partial(
        pl.kernel,
        mesh=mesh,
        out_type=jax.ShapeDtypeStruct((_NTOK, _D), jnp.float32),
        scratch_types=[
            pltpu.VMEM((_CHUNK,), jnp.int32),
            pltpu.VMEM((_CHUNK,), jnp.int32),
            pltpu.VMEM((_CHUNK, _D), jnp.float32),
            pltpu.VMEM((_S * _NSEG, _D), jnp.float32),
            pltpu.VMEM((_D,), jnp.float32),
            pltpu.VMEM((_D,), jnp.float32),
            pltpu.SemaphoreType.DMA,
        ],
        compiler_params=pltpu.CompilerParams(
            use_tc_tiling_on_sc=False, needs_layout_passes=False
        ),
    )
    def fused_k(idx_hbm, seg_hbm, table_hbm, pps_hbm, gamma_hbm, beta_hbm,
                out_hbm, idx_v, seg_v, rows_v, pps_v, gamma_v, beta_v, sem):
        wid = lax.axis_index("s") * _NC + lax.axis_index("c")
        base0 = wid * _TPW
        pltpu.sync_copy(pps_hbm, pps_v)
        pltpu.sync_copy(gamma_hbm, gamma_v)
        pltpu.sync_copy(beta_hbm, beta_v)
        gam = [gamma_v[pl.ds(16 * j, 16)] for j in range(4)]
        bet = [beta_v[pl.ds(16 * j, 16)] for j in range(4)]

        def chunk_body(c, _):
            base = base0 + c * _CHUNK
            pltpu.sync_copy(idx_hbm.at[pl.ds(base, _CHUNK)], idx_v)
            pltpu.sync_copy(seg_hbm.at[pl.ds(base, _CHUNK)], seg_v)
            copies = [
                pltpu.async_copy(
                    table_hbm.at[idx_v.at[pl.ds(j * _GSUB, _GSUB)]],
                    rows_v.at[pl.ds(j * _GSUB, _GSUB)],
                    sem,
                )
                for j in range(_CHUNK // _GSUB)
            ]
            for cp in copies:
                cp.wait()

            def grp_body(g, _):
                t0 = g * 16
                segs = seg_v[pl.ds(t0, 16)]
                svec = lax.rem(base + t0 + lax.iota(jnp.int32, 16), _S)
                cidv = svec * _NSEG + segs
                for i in range(16):
                    t = t0 + i
                    cid = cidv[i]
                    e = [
                        rows_v[t, pl.ds(16 * j, 16)] + pps_v[cid, pl.ds(16 * j, 16)]
                        for j in range(4)
                    ]
                    sumv = (e[0] + e[1]) + (e[2] + e[3])
                    sqv = (e[0] * e[0] + e[1] * e[1]) + (e[2] * e[2] + e[3] * e[3])
                    mean = jnp.sum(sumv) * (1.0 / _D)
                    var = jnp.sum(sqv) * (1.0 / _D) - mean * mean
                    x = var + 1e-6
                    iv = lax.bitcast_convert_type(x, jnp.int32)
                    iv = 0x5F3759DF - lax.shift_right_arithmetic(iv, 1)
                    y = lax.bitcast_convert_type(iv, jnp.float32)
                    y = y * (1.5 - 0.5 * x * y * y)
                    y = y * (1.5 - 0.5 * x * y * y)
                    y = y * (1.5 - 0.5 * x * y * y)
                    for j in range(4):
                        rows_v[t, pl.ds(16 * j, 16)] = (e[j] - mean) * y * gam[j] + bet[j]
                return ()

            lax.fori_loop(0, _CHUNK // 16, grp_body, ())
            pltpu.sync_copy(rows_v, out_hbm.at[pl.ds(base, _CHUNK)])
            return ()

        lax.fori_loop(0, _NCHUNK, chunk_body, ())

    return fused_k


_BB = 8


def _tc_fuse_body(rows_ref, seg_ref, pos_ref, se_ref, gamma_ref, beta_ref, out_ref):
    x = rows_ref[...]
    g = seg_ref[...][:, :, None]
    e = x + pos_ref[...][None, :, :]
    se = se_ref[...]
    for k in range(_NSEG):
        e = e + jnp.where(g == k, se[k : k + 1][None], 0.0)
    mean = jnp.mean(e, axis=-1, keepdims=True)
    var = jnp.mean((e - mean) ** 2, axis=-1, keepdims=True)
    y = (e - mean) * lax.rsqrt(var + 1e-6)
    out_ref[...] = gamma_ref[...][None] * y + beta_ref[...][None]


def _tc_fuse(rows, seg, pos, se, gamma, beta):
    return pl.pallas_call(
        _tc_fuse_body,
        grid=(_B // _BB,),
        in_specs=[
            pl.BlockSpec((_BB, _S, _D), lambda i: (i, 0, 0)),
            pl.BlockSpec((_BB, _S), lambda i: (i, 0)),
            pl.BlockSpec((_S, _D), lambda i: (0, 0)),
            pl.BlockSpec((_NSEG, _D), lambda i: (0, 0)),
            pl.BlockSpec((1, _D), lambda i: (0, 0)),
            pl.BlockSpec((1, _D), lambda i: (0, 0)),
        ],
        out_specs=pl.BlockSpec((_BB, _S, _D), lambda i: (i, 0, 0)),
        out_shape=jax.ShapeDtypeStruct((_B, _S, _D), jnp.float32),
    )(rows, seg, pos, se, gamma, beta)


def _stream(src, seg, word_emb, pos_emb, seg_emb, gamma, beta):
    idx = src.reshape(-1).astype(jnp.int32)
    segf = seg.reshape(-1).astype(jnp.int32)
    pps = (pos_emb[:_S, None, :] + seg_emb[None, :, :]).reshape(_S * _NSEG, _D)
    out = _make_sc_fused()(idx, segf, word_emb, pps, gamma, beta)
    return out.reshape(_B, _S, _D)


def kernel(src_0, src_1, seg_0, seg_1,
           word_emb_0, pos_emb_0, seg_emb_0, gamma_0, beta_0,
           word_emb_1, pos_emb_1, seg_emb_1, gamma_1, beta_1):
    out0 = _stream(src_0, seg_0, word_emb_0, pos_emb_0, seg_emb_0, gamma_0, beta_0)
    out1 = _stream(src_1, seg_1, word_emb_1, pos_emb_1, seg_emb_1, gamma_1, beta_1)
    return (out0, out1)


# speedup vs baseline: 1.0671x; 1.0671x over previous
import functools

import jax
import jax.numpy as jnp
from jax import lax
from jax.experimental import pallas as pl
from jax.experimental.pallas import tpu as pltpu
from jax.experimental.pallas import tpu_sc as plsc

_V = 100000
_D = 64
_B = 1024
_S = 200
_NSEG = 3
_NTOK = _B * _S

_NC = 2
_NS = 16
_NW = _NC * _NS
_TPW = _NTOK // _NW
_CHUNK = 640
_NCHUNK = _TPW // _CHUNK
_GSUB = 128


@functools.cache
def _make_sc_gather():
    mesh = plsc.VectorSubcoreMesh(core_axis_name="c", subcore_axis_name="s")

    @functools.partial(
        pl.kernel,
        mesh=mesh,
        out_type=jax.ShapeDtypeStruct((_NTOK, _D), jnp.float32),
        scratch_types=[
            pltpu.VMEM((_CHUNK,), jnp.int32),
            pltpu.VMEM((_CHUNK, _D), jnp.float32),
            pltpu.SemaphoreType.DMA,
        ],
        compiler_params=pltpu.CompilerParams(use_tc_tiling_on_sc=False),
    )
    def gather_k(idx_hbm, table_hbm, out_hbm, idx_v, rows_v, sem):
        wid = lax.axis_index("s") * _NC + lax.axis_index("c")
        base0 = wid * _TPW
        for c in range(_NCHUNK):
            base = base0 + c * _CHUNK
            pltpu.sync_copy(idx_hbm.at[pl.ds(base, _CHUNK)], idx_v)
            copies = []
            for j in range(_CHUNK // _GSUB):
                copies.append(
                    pltpu.async_copy(
                        table_hbm.at[idx_v.at[pl.ds(j * _GSUB, _GSUB)]],
                        rows_v.at[pl.ds(j * _GSUB, _GSUB)],
                        sem,
                    )
                )
            for cp in copies:
                cp.wait()
            pltpu.sync_copy(rows_v, out_hbm.at[pl.ds(base, _CHUNK)])

    return gather_k


@functools.cache
def _make_sc_fused():
    mesh = plsc.VectorSubcoreMesh(core_axis_name="c", subcore_axis_name="s")

    @functools.partial(
        pl.kernel,
        mesh=mesh,
        out_type=jax.ShapeDtypeStruct((_NTOK, _D), jnp.float32),
        scratch_types=[
            pltpu.VMEM((_CHUNK,), jnp.int32),
            pltpu.VMEM((_CHUNK,), jnp.int32),
            pltpu.VMEM((_CHUNK, _D), jnp.float32),
            pltpu.VMEM((_S * _NSEG, _D), jnp.float32),
            pltpu.VMEM((_D,), jnp.float32),
            pltpu.VMEM((_D,), jnp.float32),
            pltpu.SemaphoreType.DMA,
        ],
        compiler_params=pltpu.CompilerParams(
            use_tc_tiling_on_sc=False, needs_layout_passes=False
        ),
    )
    def fused_k(idx_hbm, seg_hbm, table_hbm, pps_hbm, gamma_hbm, beta_hbm,
                out_hbm, idx_v, seg_v, rows_v, pps_v, gamma_v, beta_v, sem):
        wid = lax.axis_index("s") * _NC + lax.axis_index("c")
        base0 = wid * _TPW
        pltpu.sync_copy(pps_hbm, pps_v)
        pltpu.sync_copy(gamma_hbm, gamma_v)
        pltpu.sync_copy(beta_hbm, beta_v)
        gam = [gamma_v[pl.ds(16 * j, 16)] for j in range(4)]
        bet = [beta_v[pl.ds(16 * j, 16)] for j in range(4)]

        def chunk_body(c, _):
            base = base0 + c * _CHUNK
            pltpu.sync_copy(idx_hbm.at[pl.ds(base, _CHUNK)], idx_v)
            pltpu.sync_copy(seg_hbm.at[pl.ds(base, _CHUNK)], seg_v)
            copies = [
                pltpu.async_copy(
                    table_hbm.at[idx_v.at[pl.ds(j * _GSUB, _GSUB)]],
                    rows_v.at[pl.ds(j * _GSUB, _GSUB)],
                    sem,
                )
                for j in range(_CHUNK // _GSUB)
            ]
            for cp in copies:
                cp.wait()

            def grp_body(g, _):
                t0 = g * 16
                segs = seg_v[pl.ds(t0, 16)]
                svec = lax.rem(base + t0 + lax.iota(jnp.int32, 16), _S)
                cidv = svec * _NSEG + segs
                for i in range(16):
                    t = t0 + i
                    cid = cidv[i]
                    e = [
                        rows_v[t, pl.ds(16 * j, 16)] + pps_v[cid, pl.ds(16 * j, 16)]
                        for j in range(4)
                    ]
                    sumv = (e[0] + e[1]) + (e[2] + e[3])
                    sqv = (e[0] * e[0] + e[1] * e[1]) + (e[2] * e[2] + e[3] * e[3])
                    mean = jnp.sum(sumv) * (1.0 / _D)
                    var = jnp.sum(sqv) * (1.0 / _D) - mean * mean
                    x = var + 1e-6
                    iv = lax.bitcast_convert_type(x, jnp.int32)
                    iv = 0x5F3759DF - lax.shift_right_arithmetic(iv, 1)
                    y = lax.bitcast_convert_type(iv, jnp.float32)
                    y = y * (1.5 - 0.5 * x * y * y)
                    y = y * (1.5 - 0.5 * x * y * y)
                    y = y * (1.5 - 0.5 * x * y * y)
                    for j in range(4):
                        rows_v[t, pl.ds(16 * j, 16)] = (e[j] - mean) * y * gam[j] + bet[j]
                return ()

            lax.fori_loop(0, _CHUNK // 16, grp_body, ())
            pltpu.sync_copy(rows_v, out_hbm.at[pl.ds(base, _CHUNK)])
            return ()

        lax.fori_loop(0, _NCHUNK, chunk_body, ())

    return fused_k


_BB = 8


def _tc_fuse_body(rows_ref, seg_ref, pos_ref, se_ref, gamma_ref, beta_ref, out_ref):
    x = rows_ref[...]
    g = seg_ref[...][:, :, None]
    e = x + pos_ref[...][None, :, :]
    se = se_ref[...]
    for k in range(_NSEG):
        e = e + jnp.where(g == k, se[k : k + 1][None], 0.0)
    mean = jnp.mean(e, axis=-1, keepdims=True)
    var = jnp.mean((e - mean) ** 2, axis=-1, keepdims=True)
    y = (e - mean) * lax.rsqrt(var + 1e-6)
    out_ref[...] = gamma_ref[...][None] * y + beta_ref[...][None]


def _tc_fuse(rows, seg, pos, se, gamma, beta):
    return pl.pallas_call(
        _tc_fuse_body,
        grid=(_B // _BB,),
        in_specs=[
            pl.BlockSpec((_BB, _S, _D), lambda i: (i, 0, 0)),
            pl.BlockSpec((_BB, _S), lambda i: (i, 0)),
            pl.BlockSpec((_S, _D), lambda i: (0, 0)),
            pl.BlockSpec((_NSEG, _D), lambda i: (0, 0)),
            pl.BlockSpec((1, _D), lambda i: (0, 0)),
            pl.BlockSpec((1, _D), lambda i: (0, 0)),
        ],
        out_specs=pl.BlockSpec((_BB, _S, _D), lambda i: (i, 0, 0)),
        out_shape=jax.ShapeDtypeStruct((_B, _S, _D), jnp.float32),
    )(rows, seg, pos, se, gamma, beta)


_NP = _NTOK // 2
_BP = 800
_D2 = 2 * _D


def _tc_fuse2_body(x_ref, pos_ref, oh_ref, ptab_ref, k_ref, gamma_ref, beta_ref, o_ref):
    dn = (((1,), (0,)), ((), ()))
    hi = lax.Precision.HIGHEST
    x = x_ref[...]
    segc = lax.dot_general(oh_ref[...], ptab_ref[...], dn, precision=hi)
    e = x + pos_ref[...] + segc
    k = k_ref[...]
    m = lax.dot_general(e, k, dn, precision=hi)
    q = lax.dot_general(e * e, k, dn, precision=hi)
    y = (e - m) * lax.rsqrt(q - m * m + 1e-6)
    o_ref[...] = y * gamma_ref[...] + beta_ref[...]


def _tc_fuse2(rows2, pos_tiled, oh, ptab, kmat, gamma2, beta2):
    return pl.pallas_call(
        _tc_fuse2_body,
        grid=(_NP // _BP,),
        in_specs=[
            pl.BlockSpec((_BP, _D2), lambda i: (i, 0)),
            pl.BlockSpec((_BP, _D2), lambda i: (0, 0)),
            pl.BlockSpec((_BP, 16), lambda i: (i, 0)),
            pl.BlockSpec((16, _D2), lambda i: (0, 0)),
            pl.BlockSpec((_D2, _D2), lambda i: (0, 0)),
            pl.BlockSpec((1, _D2), lambda i: (0, 0)),
            pl.BlockSpec((1, _D2), lambda i: (0, 0)),
        ],
        out_specs=pl.BlockSpec((_BP, _D2), lambda i: (i, 0)),
        out_shape=jax.ShapeDtypeStruct((_NP, _D2), jnp.float32),
    )(rows2, pos_tiled, oh, ptab, kmat, gamma2, beta2)


def _stream(src, seg, word_emb, pos_emb, seg_emb, gamma, beta):
    idx = src.reshape(-1).astype(jnp.int32)
    rows = _make_sc_gather()(idx, word_emb)
    rows2 = rows.reshape(_NP, _D2)
    pos_pairs = pos_emb[:_S].reshape(_S // 2, _D2)
    pos_tiled = jnp.tile(pos_pairs, (_BP // (_S // 2), 1))
    sp = seg.astype(jnp.int32).reshape(_NP, 2)
    code = sp[:, 0] * _NSEG + sp[:, 1]
    oh = jax.nn.one_hot(code, 16, dtype=jnp.float32)
    ptab = jnp.zeros((16, _D2), jnp.float32).at[: _NSEG * _NSEG].set(
        jnp.concatenate(
            [jnp.repeat(seg_emb, _NSEG, axis=0), jnp.tile(seg_emb, (_NSEG, 1))], axis=1
        )
    )
    half = jnp.arange(_D2) // _D
    kmat = jnp.where(half[:, None] == half[None, :], 1.0 / _D, 0.0).astype(jnp.float32)
    gamma2 = jnp.tile(gamma, 2).reshape(1, _D2)
    beta2 = jnp.tile(beta, 2).reshape(1, _D2)
    out2 = _tc_fuse2(rows2, pos_tiled, oh, ptab, kmat, gamma2, beta2)
    return out2.reshape(_B, _S, _D)


def kernel(src_0, src_1, seg_0, seg_1,
           word_emb_0, pos_emb_0, seg_emb_0, gamma_0, beta_0,
           word_emb_1, pos_emb_1, seg_emb_1, gamma_1, beta_1):
    out0 = _stream(src_0, seg_0, word_emb_0, pos_emb_0, seg_emb_0, gamma_0, beta_0)
    out1 = _stream(src_1, seg_1, word_emb_1, pos_emb_1, seg_emb_1, gamma_1, beta_1)
    return (out0, out1)


# speedup vs baseline: 1.2043x; 1.1287x over previous
import functools

import jax
import jax.numpy as jnp
from jax import lax
from jax.experimental import pallas as pl
from jax.experimental.pallas import tpu as pltpu
from jax.experimental.pallas import tpu_sc as plsc

_V = 100000
_D = 64
_B = 1024
_S = 200
_NSEG = 3
_NTOK = _B * _S

_NC = 2
_NS = 16
_NW = _NC * _NS
_TPW = _NTOK // _NW
_CHUNK = 640
_NCHUNK = _TPW // _CHUNK
_GSUB = 128


@functools.cache
def _make_sc_gather():
    mesh = plsc.VectorSubcoreMesh(core_axis_name="c", subcore_axis_name="s")

    @functools.partial(
        pl.kernel,
        mesh=mesh,
        out_type=jax.ShapeDtypeStruct((_NTOK, _D), jnp.float32),
        scratch_types=[
            pltpu.VMEM((_CHUNK,), jnp.int32),
            pltpu.VMEM((_CHUNK, _D), jnp.float32),
            pltpu.SemaphoreType.DMA,
        ],
        compiler_params=pltpu.CompilerParams(use_tc_tiling_on_sc=False),
    )
    def gather_k(idx_hbm, table_hbm, out_hbm, idx_v, rows_v, sem):
        wid = lax.axis_index("s") * _NC + lax.axis_index("c")
        base0 = wid * _TPW
        for c in range(_NCHUNK):
            base = base0 + c * _CHUNK
            pltpu.sync_copy(idx_hbm.at[pl.ds(base, _CHUNK)], idx_v)
            copies = []
            for j in range(_CHUNK // _GSUB):
                copies.append(
                    pltpu.async_copy(
                        table_hbm.at[idx_v.at[pl.ds(j * _GSUB, _GSUB)]],
                        rows_v.at[pl.ds(j * _GSUB, _GSUB)],
                        sem,
                    )
                )
            for cp in copies:
                cp.wait()
            pltpu.sync_copy(rows_v, out_hbm.at[pl.ds(base, _CHUNK)])

    return gather_k


@functools.cache
def _make_sc_fused():
    mesh = plsc.VectorSubcoreMesh(core_axis_name="c", subcore_axis_name="s")

    @functools.partial(
        pl.kernel,
        mesh=mesh,
        out_type=jax.ShapeDtypeStruct((_NTOK, _D), jnp.float32),
        scratch_types=[
            pltpu.VMEM((_CHUNK,), jnp.int32),
            pltpu.VMEM((_CHUNK,), jnp.int32),
            pltpu.VMEM((_CHUNK, _D), jnp.float32),
            pltpu.VMEM((_S * _NSEG, _D), jnp.float32),
            pltpu.VMEM((_D,), jnp.float32),
            pltpu.VMEM((_D,), jnp.float32),
            pltpu.SemaphoreType.DMA,
        ],
        compiler_params=pltpu.CompilerParams(
            use_tc_tiling_on_sc=False, needs_layout_passes=False
        ),
    )
    def fused_k(idx_hbm, seg_hbm, table_hbm, pps_hbm, gamma_hbm, beta_hbm,
                out_hbm, idx_v, seg_v, rows_v, pps_v, gamma_v, beta_v, sem):
        wid = lax.axis_index("s") * _NC + lax.axis_index("c")
        base0 = wid * _TPW
        pltpu.sync_copy(pps_hbm, pps_v)
        pltpu.sync_copy(gamma_hbm, gamma_v)
        pltpu.sync_copy(beta_hbm, beta_v)
        gam = [gamma_v[pl.ds(16 * j, 16)] for j in range(4)]
        bet = [beta_v[pl.ds(16 * j, 16)] for j in range(4)]

        def chunk_body(c, _):
            base = base0 + c * _CHUNK
            pltpu.sync_copy(idx_hbm.at[pl.ds(base, _CHUNK)], idx_v)
            pltpu.sync_copy(seg_hbm.at[pl.ds(base, _CHUNK)], seg_v)
            copies = [
                pltpu.async_copy(
                    table_hbm.at[idx_v.at[pl.ds(j * _GSUB, _GSUB)]],
                    rows_v.at[pl.ds(j * _GSUB, _GSUB)],
                    sem,
                )
                for j in range(_CHUNK // _GSUB)
            ]
            for cp in copies:
                cp.wait()

            def grp_body(g, _):
                t0 = g * 16
                segs = seg_v[pl.ds(t0, 16)]
                svec = lax.rem(base + t0 + lax.iota(jnp.int32, 16), _S)
                cidv = svec * _NSEG + segs
                for i in range(16):
                    t = t0 + i
                    cid = cidv[i]
                    e = [
                        rows_v[t, pl.ds(16 * j, 16)] + pps_v[cid, pl.ds(16 * j, 16)]
                        for j in range(4)
                    ]
                    sumv = (e[0] + e[1]) + (e[2] + e[3])
                    sqv = (e[0] * e[0] + e[1] * e[1]) + (e[2] * e[2] + e[3] * e[3])
                    mean = jnp.sum(sumv) * (1.0 / _D)
                    var = jnp.sum(sqv) * (1.0 / _D) - mean * mean
                    x = var + 1e-6
                    iv = lax.bitcast_convert_type(x, jnp.int32)
                    iv = 0x5F3759DF - lax.shift_right_arithmetic(iv, 1)
                    y = lax.bitcast_convert_type(iv, jnp.float32)
                    y = y * (1.5 - 0.5 * x * y * y)
                    y = y * (1.5 - 0.5 * x * y * y)
                    y = y * (1.5 - 0.5 * x * y * y)
                    for j in range(4):
                        rows_v[t, pl.ds(16 * j, 16)] = (e[j] - mean) * y * gam[j] + bet[j]
                return ()

            lax.fori_loop(0, _CHUNK // 16, grp_body, ())
            pltpu.sync_copy(rows_v, out_hbm.at[pl.ds(base, _CHUNK)])
            return ()

        lax.fori_loop(0, _NCHUNK, chunk_body, ())

    return fused_k


_BB = 8


def _tc_fuse_body(rows_ref, seg_ref, pos_ref, se_ref, gamma_ref, beta_ref, out_ref):
    x = rows_ref[...]
    g = seg_ref[...][:, :, None]
    e = x + pos_ref[...][None, :, :]
    se = se_ref[...]
    for k in range(_NSEG):
        e = e + jnp.where(g == k, se[k : k + 1][None], 0.0)
    mean = jnp.mean(e, axis=-1, keepdims=True)
    var = jnp.mean((e - mean) ** 2, axis=-1, keepdims=True)
    y = (e - mean) * lax.rsqrt(var + 1e-6)
    out_ref[...] = gamma_ref[...][None] * y + beta_ref[...][None]


def _tc_fuse(rows, seg, pos, se, gamma, beta):
    return pl.pallas_call(
        _tc_fuse_body,
        grid=(_B // _BB,),
        in_specs=[
            pl.BlockSpec((_BB, _S, _D), lambda i: (i, 0, 0)),
            pl.BlockSpec((_BB, _S), lambda i: (i, 0)),
            pl.BlockSpec((_S, _D), lambda i: (0, 0)),
            pl.BlockSpec((_NSEG, _D), lambda i: (0, 0)),
            pl.BlockSpec((1, _D), lambda i: (0, 0)),
            pl.BlockSpec((1, _D), lambda i: (0, 0)),
        ],
        out_specs=pl.BlockSpec((_BB, _S, _D), lambda i: (i, 0, 0)),
        out_shape=jax.ShapeDtypeStruct((_B, _S, _D), jnp.float32),
    )(rows, seg, pos, se, gamma, beta)


_NP = _NTOK // 2
_BP = 800
_D2 = 2 * _D


def _tc_fuse2_body(x_ref, pos_ref, oh_ref, ptab_ref, k_ref, gamma_ref, beta_ref, o_ref):
    dn = (((1,), (0,)), ((), ()))
    hi = lax.Precision.DEFAULT
    x = x_ref[...]
    segc = lax.dot_general(oh_ref[...], ptab_ref[...], dn, precision=hi)
    e = x + pos_ref[...] + segc
    k = k_ref[...]
    m = lax.dot_general(e, k, dn, precision=hi)
    q = lax.dot_general(e * e, k, dn, precision=hi)
    y = (e - m) * lax.rsqrt(q - m * m + 1e-6)
    o_ref[...] = y * gamma_ref[...] + beta_ref[...]


def _tc_fuse2(rows2, pos_tiled, oh, ptab, kmat, gamma2, beta2):
    return pl.pallas_call(
        _tc_fuse2_body,
        grid=(_NP // _BP,),
        in_specs=[
            pl.BlockSpec((_BP, _D2), lambda i: (i, 0)),
            pl.BlockSpec((_BP, _D2), lambda i: (0, 0)),
            pl.BlockSpec((_BP, 16), lambda i: (i, 0)),
            pl.BlockSpec((16, _D2), lambda i: (0, 0)),
            pl.BlockSpec((_D2, _D2), lambda i: (0, 0)),
            pl.BlockSpec((1, _D2), lambda i: (0, 0)),
            pl.BlockSpec((1, _D2), lambda i: (0, 0)),
        ],
        out_specs=pl.BlockSpec((_BP, _D2), lambda i: (i, 0)),
        out_shape=jax.ShapeDtypeStruct((_NP, _D2), jnp.float32),
    )(rows2, pos_tiled, oh, ptab, kmat, gamma2, beta2)


def _stream(src, seg, word_emb, pos_emb, seg_emb, gamma, beta):
    idx = src.reshape(-1).astype(jnp.int32)
    rows2 = _make_sc_gather()(idx, word_emb).reshape(_NP, _D2)
    pos_pairs = pos_emb[:_S].reshape(_S // 2, _D2)
    pos_tiled = jnp.tile(pos_pairs, (_BP // (_S // 2), 1))
    sp = seg.astype(jnp.int32).reshape(_NP, 2)
    code = sp[:, 0] * _NSEG + sp[:, 1]
    oh = jax.nn.one_hot(code, 16, dtype=jnp.float32)
    ptab = jnp.zeros((16, _D2), jnp.float32).at[: _NSEG * _NSEG].set(
        jnp.concatenate(
            [jnp.repeat(seg_emb, _NSEG, axis=0), jnp.tile(seg_emb, (_NSEG, 1))], axis=1
        )
    )
    half = jnp.arange(_D2) // _D
    kmat = jnp.where(half[:, None] == half[None, :], 1.0 / _D, 0.0).astype(jnp.float32)
    gamma2 = jnp.tile(gamma, 2).reshape(1, _D2)
    beta2 = jnp.tile(beta, 2).reshape(1, _D2)
    out2 = _tc_fuse2(rows2, pos_tiled, oh, ptab, kmat, gamma2, beta2)
    return out2.reshape(_B, _S, _D)


def kernel(src_0, src_1, seg_0, seg_1,
           word_emb_0, pos_emb_0, seg_emb_0, gamma_0, beta_0,
           word_emb_1, pos_emb_1, seg_emb_1, gamma_1, beta_1):
    out0 = _stream(src_0, seg_0, word_emb_0, pos_emb_0, seg_emb_0, gamma_0, beta_0)
    out1 = _stream(src_1, seg_1, word_emb_1, pos_emb_1, seg_emb_1, gamma_1, beta_1)
    return (out0, out1)


# speedup vs baseline: 1.3063x; 1.0846x over previous
import functools

import jax
import jax.numpy as jnp
from jax import lax
from jax.experimental import pallas as pl
from jax.experimental.pallas import tpu as pltpu
from jax.experimental.pallas import tpu_sc as plsc

_V = 100000
_D = 64
_B = 1024
_S = 200
_NSEG = 3
_NTOK = _B * _S

_NC = 2
_NS = 16
_NW = _NC * _NS
_TPW = _NTOK // _NW
_CHUNK = 640
_NCHUNK = _TPW // _CHUNK
_GSUB = 128


@functools.cache
def _make_sc_gather():
    mesh = plsc.VectorSubcoreMesh(core_axis_name="c", subcore_axis_name="s")

    @functools.partial(
        pl.kernel,
        mesh=mesh,
        out_type=jax.ShapeDtypeStruct((_NTOK, _D), jnp.float32),
        scratch_types=[
            pltpu.VMEM((_CHUNK,), jnp.int32),
            pltpu.VMEM((_CHUNK, _D), jnp.float32),
            pltpu.SemaphoreType.DMA,
        ],
        compiler_params=pltpu.CompilerParams(use_tc_tiling_on_sc=False),
    )
    def gather_k(idx_hbm, table_hbm, out_hbm, idx_v, rows_v, sem):
        wid = lax.axis_index("s") * _NC + lax.axis_index("c")
        base0 = wid * _TPW
        for c in range(_NCHUNK):
            base = base0 + c * _CHUNK
            pltpu.sync_copy(idx_hbm.at[pl.ds(base, _CHUNK)], idx_v)
            copies = []
            for j in range(_CHUNK // _GSUB):
                copies.append(
                    pltpu.async_copy(
                        table_hbm.at[idx_v.at[pl.ds(j * _GSUB, _GSUB)]],
                        rows_v.at[pl.ds(j * _GSUB, _GSUB)],
                        sem,
                    )
                )
            for cp in copies:
                cp.wait()
            pltpu.sync_copy(rows_v, out_hbm.at[pl.ds(base, _CHUNK)])

    return gather_k


@functools.cache
def _make_sc_fused():
    mesh = plsc.VectorSubcoreMesh(core_axis_name="c", subcore_axis_name="s")

    @functools.partial(
        pl.kernel,
        mesh=mesh,
        out_type=jax.ShapeDtypeStruct((_NTOK, _D), jnp.float32),
        scratch_types=[
            pltpu.VMEM((_CHUNK,), jnp.int32),
            pltpu.VMEM((_CHUNK,), jnp.int32),
            pltpu.VMEM((_CHUNK, _D), jnp.float32),
            pltpu.VMEM((_S * _NSEG, _D), jnp.float32),
            pltpu.VMEM((_D,), jnp.float32),
            pltpu.VMEM((_D,), jnp.float32),
            pltpu.SemaphoreType.DMA,
        ],
        compiler_params=pltpu.CompilerParams(
            use_tc_tiling_on_sc=False, needs_layout_passes=False
        ),
    )
    def fused_k(idx_hbm, seg_hbm, table_hbm, pps_hbm, gamma_hbm, beta_hbm,
                out_hbm, idx_v, seg_v, rows_v, pps_v, gamma_v, beta_v, sem):
        wid = lax.axis_index("s") * _NC + lax.axis_index("c")
        base0 = wid * _TPW
        pltpu.sync_copy(pps_hbm, pps_v)
        pltpu.sync_copy(gamma_hbm, gamma_v)
        pltpu.sync_copy(beta_hbm, beta_v)
        gam = [gamma_v[pl.ds(16 * j, 16)] for j in range(4)]
        bet = [beta_v[pl.ds(16 * j, 16)] for j in range(4)]

        def chunk_body(c, _):
            base = base0 + c * _CHUNK
            pltpu.sync_copy(idx_hbm.at[pl.ds(base, _CHUNK)], idx_v)
            pltpu.sync_copy(seg_hbm.at[pl.ds(base, _CHUNK)], seg_v)
            copies = [
                pltpu.async_copy(
                    table_hbm.at[idx_v.at[pl.ds(j * _GSUB, _GSUB)]],
                    rows_v.at[pl.ds(j * _GSUB, _GSUB)],
                    sem,
                )
                for j in range(_CHUNK // _GSUB)
            ]
            for cp in copies:
                cp.wait()

            def grp_body(g, _):
                t0 = g * 16
                segs = seg_v[pl.ds(t0, 16)]
                svec = lax.rem(base + t0 + lax.iota(jnp.int32, 16), _S)
                cidv = svec * _NSEG + segs
                for i in range(16):
                    t = t0 + i
                    cid = cidv[i]
                    e = [
                        rows_v[t, pl.ds(16 * j, 16)] + pps_v[cid, pl.ds(16 * j, 16)]
                        for j in range(4)
                    ]
                    sumv = (e[0] + e[1]) + (e[2] + e[3])
                    sqv = (e[0] * e[0] + e[1] * e[1]) + (e[2] * e[2] + e[3] * e[3])
                    mean = jnp.sum(sumv) * (1.0 / _D)
                    var = jnp.sum(sqv) * (1.0 / _D) - mean * mean
                    x = var + 1e-6
                    iv = lax.bitcast_convert_type(x, jnp.int32)
                    iv = 0x5F3759DF - lax.shift_right_arithmetic(iv, 1)
                    y = lax.bitcast_convert_type(iv, jnp.float32)
                    y = y * (1.5 - 0.5 * x * y * y)
                    y = y * (1.5 - 0.5 * x * y * y)
                    y = y * (1.5 - 0.5 * x * y * y)
                    for j in range(4):
                        rows_v[t, pl.ds(16 * j, 16)] = (e[j] - mean) * y * gam[j] + bet[j]
                return ()

            lax.fori_loop(0, _CHUNK // 16, grp_body, ())
            pltpu.sync_copy(rows_v, out_hbm.at[pl.ds(base, _CHUNK)])
            return ()

        lax.fori_loop(0, _NCHUNK, chunk_body, ())

    return fused_k


_BB = 8


def _tc_fuse_body(rows_ref, seg_ref, pos_ref, se_ref, gamma_ref, beta_ref, out_ref):
    x = rows_ref[...]
    g = seg_ref[...][:, :, None]
    e = x + pos_ref[...][None, :, :]
    se = se_ref[...]
    for k in range(_NSEG):
        e = e + jnp.where(g == k, se[k : k + 1][None], 0.0)
    mean = jnp.mean(e, axis=-1, keepdims=True)
    var = jnp.mean((e - mean) ** 2, axis=-1, keepdims=True)
    y = (e - mean) * lax.rsqrt(var + 1e-6)
    out_ref[...] = gamma_ref[...][None] * y + beta_ref[...][None]


def _tc_fuse(rows, seg, pos, se, gamma, beta):
    return pl.pallas_call(
        _tc_fuse_body,
        grid=(_B // _BB,),
        in_specs=[
            pl.BlockSpec((_BB, _S, _D), lambda i: (i, 0, 0)),
            pl.BlockSpec((_BB, _S), lambda i: (i, 0)),
            pl.BlockSpec((_S, _D), lambda i: (0, 0)),
            pl.BlockSpec((_NSEG, _D), lambda i: (0, 0)),
            pl.BlockSpec((1, _D), lambda i: (0, 0)),
            pl.BlockSpec((1, _D), lambda i: (0, 0)),
        ],
        out_specs=pl.BlockSpec((_BB, _S, _D), lambda i: (i, 0, 0)),
        out_shape=jax.ShapeDtypeStruct((_B, _S, _D), jnp.float32),
    )(rows, seg, pos, se, gamma, beta)


_NP = _NTOK // 2
_BP = 800
_D2 = 2 * _D


def _tc_fuse2_body(x_ref, pos_ref, oh_ref, ptab_ref, k_ref, gamma_ref, beta_ref, o_ref):
    dn = (((1,), (0,)), ((), ()))
    hi = lax.Precision.DEFAULT
    x = x_ref[...]
    segc = lax.dot_general(
        oh_ref[0], ptab_ref[...], (((0,), (0,)), ((), ())), precision=hi
    )
    e = x + pos_ref[...] + segc
    k = k_ref[...]
    m = lax.dot_general(e, k, dn, precision=hi)
    q = lax.dot_general(e * e, k, dn, precision=hi)
    y = (e - m) * lax.rsqrt(q - m * m + 1e-6)
    o_ref[...] = y * gamma_ref[...] + beta_ref[...]


def _tc_fuse2(rows2, pos_tiled, oh, ptab, kmat, gamma2, beta2):
    return pl.pallas_call(
        _tc_fuse2_body,
        grid=(_NP // _BP,),
        in_specs=[
            pl.BlockSpec((_BP, _D2), lambda i: (i, 0)),
            pl.BlockSpec((_BP, _D2), lambda i: (0, 0)),
            pl.BlockSpec((1, 16, _BP), lambda i: (i, 0, 0)),
            pl.BlockSpec((16, _D2), lambda i: (0, 0)),
            pl.BlockSpec((_D2, _D2), lambda i: (0, 0)),
            pl.BlockSpec((1, _D2), lambda i: (0, 0)),
            pl.BlockSpec((1, _D2), lambda i: (0, 0)),
        ],
        out_specs=pl.BlockSpec((_BP, _D2), lambda i: (i, 0)),
        out_shape=jax.ShapeDtypeStruct((_NP, _D2), jnp.float32),
    )(rows2, pos_tiled, oh, ptab, kmat, gamma2, beta2)


def _stream(src, seg, word_emb, pos_emb, seg_emb, gamma, beta):
    idx = src.reshape(-1).astype(jnp.int32)
    rows2 = _make_sc_gather()(idx, word_emb).reshape(_NP, _D2)
    pos_pairs = pos_emb[:_S].reshape(_S // 2, _D2)
    pos_tiled = jnp.tile(pos_pairs, (_BP // (_S // 2), 1))
    sp = seg.astype(jnp.int32).reshape(_NP, 2)
    code = sp[:, 0] * _NSEG + sp[:, 1]
    oh = jax.nn.one_hot(code, 16, dtype=jnp.float32, axis=0)
    oh = oh.reshape(16, _NP // _BP, _BP).transpose(1, 0, 2)
    ptab = jnp.zeros((16, _D2), jnp.float32).at[: _NSEG * _NSEG].set(
        jnp.concatenate(
            [jnp.repeat(seg_emb, _NSEG, axis=0), jnp.tile(seg_emb, (_NSEG, 1))], axis=1
        )
    )
    half = jnp.arange(_D2) // _D
    kmat = jnp.where(half[:, None] == half[None, :], 1.0 / _D, 0.0).astype(jnp.float32)
    gamma2 = jnp.tile(gamma, 2).reshape(1, _D2)
    beta2 = jnp.tile(beta, 2).reshape(1, _D2)
    out2 = _tc_fuse2(rows2, pos_tiled, oh, ptab, kmat, gamma2, beta2)
    return out2.reshape(_B, _S, _D)


def kernel(src_0, src_1, seg_0, seg_1,
           word_emb_0, pos_emb_0, seg_emb_0, gamma_0, beta_0,
           word_emb_1, pos_emb_1, seg_emb_1, gamma_1, beta_1):
    out0 = _stream(src_0, seg_0, word_emb_0, pos_emb_0, seg_emb_0, gamma_0, beta_0)
    out1 = _stream(src_1, seg_1, word_emb_1, pos_emb_1, seg_emb_1, gamma_1, beta_1)
    return (out0, out1)


# speedup vs baseline: 1.4398x; 1.1022x over previous
import functools

import jax
import jax.numpy as jnp
from jax import lax
from jax.experimental import pallas as pl
from jax.experimental.pallas import tpu as pltpu
from jax.experimental.pallas import tpu_sc as plsc

_V = 100000
_D = 64
_B = 1024
_S = 200
_NSEG = 3
_NTOK = _B * _S

_NC = 2
_NS = 16
_NW = _NC * _NS
_TPW = _NTOK // _NW
_CHUNK = 640
_NCHUNK = _TPW // _CHUNK
_GSUB = 128


@functools.cache
def _make_sc_gather():
    mesh = plsc.VectorSubcoreMesh(core_axis_name="c", subcore_axis_name="s")

    @functools.partial(
        pl.kernel,
        mesh=mesh,
        out_type=jax.ShapeDtypeStruct((_NTOK, _D), jnp.float32),
        scratch_types=[
            pltpu.VMEM((_CHUNK,), jnp.int32),
            pltpu.VMEM((_CHUNK, _D), jnp.float32),
            pltpu.SemaphoreType.DMA,
        ],
        compiler_params=pltpu.CompilerParams(use_tc_tiling_on_sc=False),
    )
    def gather_k(idx_hbm, table_hbm, out_hbm, idx_v, rows_v, sem):
        wid = lax.axis_index("s") * _NC + lax.axis_index("c")
        base0 = wid * _TPW
        for c in range(_NCHUNK):
            base = base0 + c * _CHUNK
            pltpu.sync_copy(idx_hbm.at[pl.ds(base, _CHUNK)], idx_v)
            copies = []
            for j in range(_CHUNK // _GSUB):
                copies.append(
                    pltpu.async_copy(
                        table_hbm.at[idx_v.at[pl.ds(j * _GSUB, _GSUB)]],
                        rows_v.at[pl.ds(j * _GSUB, _GSUB)],
                        sem,
                    )
                )
            for cp in copies:
                cp.wait()
            pltpu.sync_copy(rows_v, out_hbm.at[pl.ds(base, _CHUNK)])

    return gather_k


@functools.cache
def _make_sc_fused():
    mesh = plsc.VectorSubcoreMesh(core_axis_name="c", subcore_axis_name="s")

    @functools.partial(
        pl.kernel,
        mesh=mesh,
        out_type=jax.ShapeDtypeStruct((_NTOK, _D), jnp.float32),
        scratch_types=[
            pltpu.VMEM((_CHUNK,), jnp.int32),
            pltpu.VMEM((_CHUNK,), jnp.int32),
            pltpu.VMEM((_CHUNK, _D), jnp.float32),
            pltpu.VMEM((_S * _NSEG, _D), jnp.float32),
            pltpu.VMEM((_D,), jnp.float32),
            pltpu.VMEM((_D,), jnp.float32),
            pltpu.SemaphoreType.DMA,
        ],
        compiler_params=pltpu.CompilerParams(
            use_tc_tiling_on_sc=False, needs_layout_passes=False
        ),
    )
    def fused_k(idx_hbm, seg_hbm, table_hbm, pps_hbm, gamma_hbm, beta_hbm,
                out_hbm, idx_v, seg_v, rows_v, pps_v, gamma_v, beta_v, sem):
        wid = lax.axis_index("s") * _NC + lax.axis_index("c")
        base0 = wid * _TPW
        pltpu.sync_copy(pps_hbm, pps_v)
        pltpu.sync_copy(gamma_hbm, gamma_v)
        pltpu.sync_copy(beta_hbm, beta_v)
        gam = [gamma_v[pl.ds(16 * j, 16)] for j in range(4)]
        bet = [beta_v[pl.ds(16 * j, 16)] for j in range(4)]

        def chunk_body(c, _):
            base = base0 + c * _CHUNK
            pltpu.sync_copy(idx_hbm.at[pl.ds(base, _CHUNK)], idx_v)
            pltpu.sync_copy(seg_hbm.at[pl.ds(base, _CHUNK)], seg_v)
            copies = [
                pltpu.async_copy(
                    table_hbm.at[idx_v.at[pl.ds(j * _GSUB, _GSUB)]],
                    rows_v.at[pl.ds(j * _GSUB, _GSUB)],
                    sem,
                )
                for j in range(_CHUNK // _GSUB)
            ]
            for cp in copies:
                cp.wait()

            def grp_body(g, _):
                t0 = g * 16
                segs = seg_v[pl.ds(t0, 16)]
                svec = lax.rem(base + t0 + lax.iota(jnp.int32, 16), _S)
                cidv = svec * _NSEG + segs
                for i in range(16):
                    t = t0 + i
                    cid = cidv[i]
                    e = [
                        rows_v[t, pl.ds(16 * j, 16)] + pps_v[cid, pl.ds(16 * j, 16)]
                        for j in range(4)
                    ]
                    sumv = (e[0] + e[1]) + (e[2] + e[3])
                    sqv = (e[0] * e[0] + e[1] * e[1]) + (e[2] * e[2] + e[3] * e[3])
                    mean = jnp.sum(sumv) * (1.0 / _D)
                    var = jnp.sum(sqv) * (1.0 / _D) - mean * mean
                    x = var + 1e-6
                    iv = lax.bitcast_convert_type(x, jnp.int32)
                    iv = 0x5F3759DF - lax.shift_right_arithmetic(iv, 1)
                    y = lax.bitcast_convert_type(iv, jnp.float32)
                    y = y * (1.5 - 0.5 * x * y * y)
                    y = y * (1.5 - 0.5 * x * y * y)
                    y = y * (1.5 - 0.5 * x * y * y)
                    for j in range(4):
                        rows_v[t, pl.ds(16 * j, 16)] = (e[j] - mean) * y * gam[j] + bet[j]
                return ()

            lax.fori_loop(0, _CHUNK // 16, grp_body, ())
            pltpu.sync_copy(rows_v, out_hbm.at[pl.ds(base, _CHUNK)])
            return ()

        lax.fori_loop(0, _NCHUNK, chunk_body, ())

    return fused_k


_BB = 8


def _tc_fuse_body(rows_ref, seg_ref, pos_ref, se_ref, gamma_ref, beta_ref, out_ref):
    x = rows_ref[...]
    g = seg_ref[...][:, :, None]
    e = x + pos_ref[...][None, :, :]
    se = se_ref[...]
    for k in range(_NSEG):
        e = e + jnp.where(g == k, se[k : k + 1][None], 0.0)
    mean = jnp.mean(e, axis=-1, keepdims=True)
    var = jnp.mean((e - mean) ** 2, axis=-1, keepdims=True)
    y = (e - mean) * lax.rsqrt(var + 1e-6)
    out_ref[...] = gamma_ref[...][None] * y + beta_ref[...][None]


def _tc_fuse(rows, seg, pos, se, gamma, beta):
    return pl.pallas_call(
        _tc_fuse_body,
        grid=(_B // _BB,),
        in_specs=[
            pl.BlockSpec((_BB, _S, _D), lambda i: (i, 0, 0)),
            pl.BlockSpec((_BB, _S), lambda i: (i, 0)),
            pl.BlockSpec((_S, _D), lambda i: (0, 0)),
            pl.BlockSpec((_NSEG, _D), lambda i: (0, 0)),
            pl.BlockSpec((1, _D), lambda i: (0, 0)),
            pl.BlockSpec((1, _D), lambda i: (0, 0)),
        ],
        out_specs=pl.BlockSpec((_BB, _S, _D), lambda i: (i, 0, 0)),
        out_shape=jax.ShapeDtypeStruct((_B, _S, _D), jnp.float32),
    )(rows, seg, pos, se, gamma, beta)


_NP = _NTOK // 2
_BP = 1600
_D2 = 2 * _D


def _tc_fuse2_body(x_ref, pos_ref, oh_ref, ptab_ref, k_ref, gamma_ref, beta_ref, o_ref):
    dn = (((1,), (0,)), ((), ()))
    hi = lax.Precision.DEFAULT
    x = x_ref[...]
    segc = lax.dot_general(
        oh_ref[0], ptab_ref[...], (((0,), (0,)), ((), ())), precision=hi
    )
    e = x + pos_ref[...] + segc
    k = k_ref[...]
    m = lax.dot_general(e, k, dn, precision=hi)
    q = lax.dot_general(e * e, k, dn, precision=hi)
    y = (e - m) * lax.rsqrt(q - m * m + 1e-6)
    o_ref[...] = y * gamma_ref[...] + beta_ref[...]


def _tc_fuse2(rows2, pos_tiled, oh, ptab, kmat, gamma2, beta2):
    return pl.pallas_call(
        _tc_fuse2_body,
        grid=(_NP // _BP,),
        in_specs=[
            pl.BlockSpec((_BP, _D2), lambda i: (i, 0)),
            pl.BlockSpec((_BP, _D2), lambda i: (0, 0)),
            pl.BlockSpec((1, 16, _BP), lambda i: (i, 0, 0)),
            pl.BlockSpec((16, _D2), lambda i: (0, 0)),
            pl.BlockSpec((_D2, _D2), lambda i: (0, 0)),
            pl.BlockSpec((1, _D2), lambda i: (0, 0)),
            pl.BlockSpec((1, _D2), lambda i: (0, 0)),
        ],
        out_specs=pl.BlockSpec((_BP, _D2), lambda i: (i, 0)),
        out_shape=jax.ShapeDtypeStruct((_NP, _D2), jnp.float32),
    )(rows2, pos_tiled, oh, ptab, kmat, gamma2, beta2)


def _stream(src, seg, word_emb, pos_emb, seg_emb, gamma, beta):
    idx = src.reshape(-1).astype(jnp.int32)
    rows2 = _make_sc_gather()(idx, word_emb).reshape(_NP, _D2)
    pos_pairs = pos_emb[:_S].reshape(_S // 2, _D2)
    pos_tiled = jnp.tile(pos_pairs, (_BP // (_S // 2), 1))
    sp = seg.astype(jnp.int32).reshape(_NP, 2)
    code = sp[:, 0] * _NSEG + sp[:, 1]
    oh = jax.nn.one_hot(code, 16, dtype=jnp.float32, axis=0)
    oh = oh.reshape(16, _NP // _BP, _BP).transpose(1, 0, 2)
    ptab = jnp.zeros((16, _D2), jnp.float32).at[: _NSEG * _NSEG].set(
        jnp.concatenate(
            [jnp.repeat(seg_emb, _NSEG, axis=0), jnp.tile(seg_emb, (_NSEG, 1))], axis=1
        )
    )
    half = jnp.arange(_D2) // _D
    kmat = jnp.where(half[:, None] == half[None, :], 1.0 / _D, 0.0).astype(jnp.float32)
    gamma2 = jnp.tile(gamma, 2).reshape(1, _D2)
    beta2 = jnp.tile(beta, 2).reshape(1, _D2)
    out2 = _tc_fuse2(rows2, pos_tiled, oh, ptab, kmat, gamma2, beta2)
    return out2.reshape(_B, _S, _D)


def kernel(src_0, src_1, seg_0, seg_1,
           word_emb_0, pos_emb_0, seg_emb_0, gamma_0, beta_0,
           word_emb_1, pos_emb_1, seg_emb_1, gamma_1, beta_1):
    out0 = _stream(src_0, seg_0, word_emb_0, pos_emb_0, seg_emb_0, gamma_0, beta_0)
    out1 = _stream(src_1, seg_1, word_emb_1, pos_emb_1, seg_emb_1, gamma_1, beta_1)
    return (out0, out1)


# speedup vs baseline: 1.5174x; 1.0539x over previous
import functools

import jax
import jax.numpy as jnp
from jax import lax
from jax.experimental import pallas as pl
from jax.experimental.pallas import tpu as pltpu
from jax.experimental.pallas import tpu_sc as plsc

_V = 100000
_D = 64
_B = 1024
_S = 200
_NSEG = 3
_NTOK = _B * _S

_NC = 2
_NS = 16
_NW = _NC * _NS
_TPW = _NTOK // _NW
_CHUNK = 640
_NCHUNK = _TPW // _CHUNK
_GSUB = 128


@functools.cache
def _make_sc_gather():
    mesh = plsc.VectorSubcoreMesh(core_axis_name="c", subcore_axis_name="s")

    @functools.partial(
        pl.kernel,
        mesh=mesh,
        out_type=jax.ShapeDtypeStruct((_NTOK, _D), jnp.float32),
        scratch_types=[
            pltpu.VMEM((_CHUNK,), jnp.int32),
            pltpu.VMEM((_CHUNK, _D), jnp.float32),
            pltpu.SemaphoreType.DMA,
        ],
        compiler_params=pltpu.CompilerParams(use_tc_tiling_on_sc=False),
    )
    def gather_k(idx_hbm, table_hbm, out_hbm, idx_v, rows_v, sem):
        wid = lax.axis_index("s") * _NC + lax.axis_index("c")
        base0 = wid * _TPW
        for c in range(_NCHUNK):
            base = base0 + c * _CHUNK
            pltpu.sync_copy(idx_hbm.at[pl.ds(base, _CHUNK)], idx_v)
            copies = []
            for j in range(_CHUNK // _GSUB):
                copies.append(
                    pltpu.async_copy(
                        table_hbm.at[idx_v.at[pl.ds(j * _GSUB, _GSUB)]],
                        rows_v.at[pl.ds(j * _GSUB, _GSUB)],
                        sem,
                    )
                )
            for cp in copies:
                cp.wait()
            pltpu.sync_copy(rows_v, out_hbm.at[pl.ds(base, _CHUNK)])

    return gather_k


@functools.cache
def _make_sc_fused():
    mesh = plsc.VectorSubcoreMesh(core_axis_name="c", subcore_axis_name="s")

    @functools.partial(
        pl.kernel,
        mesh=mesh,
        out_type=jax.ShapeDtypeStruct((_NTOK, _D), jnp.float32),
        scratch_types=[
            pltpu.VMEM((_CHUNK,), jnp.int32),
            pltpu.VMEM((_CHUNK,), jnp.int32),
            pltpu.VMEM((_CHUNK, _D), jnp.float32),
            pltpu.VMEM((_S * _NSEG, _D), jnp.float32),
            pltpu.VMEM((_D,), jnp.float32),
            pltpu.VMEM((_D,), jnp.float32),
            pltpu.SemaphoreType.DMA,
        ],
        compiler_params=pltpu.CompilerParams(
            use_tc_tiling_on_sc=False, needs_layout_passes=False
        ),
    )
    def fused_k(idx_hbm, seg_hbm, table_hbm, pps_hbm, gamma_hbm, beta_hbm,
                out_hbm, idx_v, seg_v, rows_v, pps_v, gamma_v, beta_v, sem):
        wid = lax.axis_index("s") * _NC + lax.axis_index("c")
        base0 = wid * _TPW
        pltpu.sync_copy(pps_hbm, pps_v)
        pltpu.sync_copy(gamma_hbm, gamma_v)
        pltpu.sync_copy(beta_hbm, beta_v)
        gam = [gamma_v[pl.ds(16 * j, 16)] for j in range(4)]
        bet = [beta_v[pl.ds(16 * j, 16)] for j in range(4)]

        def chunk_body(c, _):
            base = base0 + c * _CHUNK
            pltpu.sync_copy(idx_hbm.at[pl.ds(base, _CHUNK)], idx_v)
            pltpu.sync_copy(seg_hbm.at[pl.ds(base, _CHUNK)], seg_v)
            copies = [
                pltpu.async_copy(
                    table_hbm.at[idx_v.at[pl.ds(j * _GSUB, _GSUB)]],
                    rows_v.at[pl.ds(j * _GSUB, _GSUB)],
                    sem,
                )
                for j in range(_CHUNK // _GSUB)
            ]
            for cp in copies:
                cp.wait()

            def grp_body(g, _):
                t0 = g * 16
                segs = seg_v[pl.ds(t0, 16)]
                svec = lax.rem(base + t0 + lax.iota(jnp.int32, 16), _S)
                cidv = svec * _NSEG + segs
                for i in range(16):
                    t = t0 + i
                    cid = cidv[i]
                    e = [
                        rows_v[t, pl.ds(16 * j, 16)] + pps_v[cid, pl.ds(16 * j, 16)]
                        for j in range(4)
                    ]
                    sumv = (e[0] + e[1]) + (e[2] + e[3])
                    sqv = (e[0] * e[0] + e[1] * e[1]) + (e[2] * e[2] + e[3] * e[3])
                    mean = jnp.sum(sumv) * (1.0 / _D)
                    var = jnp.sum(sqv) * (1.0 / _D) - mean * mean
                    x = var + 1e-6
                    iv = lax.bitcast_convert_type(x, jnp.int32)
                    iv = 0x5F3759DF - lax.shift_right_arithmetic(iv, 1)
                    y = lax.bitcast_convert_type(iv, jnp.float32)
                    y = y * (1.5 - 0.5 * x * y * y)
                    y = y * (1.5 - 0.5 * x * y * y)
                    y = y * (1.5 - 0.5 * x * y * y)
                    for j in range(4):
                        rows_v[t, pl.ds(16 * j, 16)] = (e[j] - mean) * y * gam[j] + bet[j]
                return ()

            lax.fori_loop(0, _CHUNK // 16, grp_body, ())
            pltpu.sync_copy(rows_v, out_hbm.at[pl.ds(base, _CHUNK)])
            return ()

        lax.fori_loop(0, _NCHUNK, chunk_body, ())

    return fused_k


_BB = 8


def _tc_fuse_body(rows_ref, seg_ref, pos_ref, se_ref, gamma_ref, beta_ref, out_ref):
    x = rows_ref[...]
    g = seg_ref[...][:, :, None]
    e = x + pos_ref[...][None, :, :]
    se = se_ref[...]
    for k in range(_NSEG):
        e = e + jnp.where(g == k, se[k : k + 1][None], 0.0)
    mean = jnp.mean(e, axis=-1, keepdims=True)
    var = jnp.mean((e - mean) ** 2, axis=-1, keepdims=True)
    y = (e - mean) * lax.rsqrt(var + 1e-6)
    out_ref[...] = gamma_ref[...][None] * y + beta_ref[...][None]


def _tc_fuse(rows, seg, pos, se, gamma, beta):
    return pl.pallas_call(
        _tc_fuse_body,
        grid=(_B // _BB,),
        in_specs=[
            pl.BlockSpec((_BB, _S, _D), lambda i: (i, 0, 0)),
            pl.BlockSpec((_BB, _S), lambda i: (i, 0)),
            pl.BlockSpec((_S, _D), lambda i: (0, 0)),
            pl.BlockSpec((_NSEG, _D), lambda i: (0, 0)),
            pl.BlockSpec((1, _D), lambda i: (0, 0)),
            pl.BlockSpec((1, _D), lambda i: (0, 0)),
        ],
        out_specs=pl.BlockSpec((_BB, _S, _D), lambda i: (i, 0, 0)),
        out_shape=jax.ShapeDtypeStruct((_B, _S, _D), jnp.float32),
    )(rows, seg, pos, se, gamma, beta)


_NP = _NTOK // 2
_BP = 3200
_D2 = 2 * _D


def _tc_fuse2_body(x_ref, pos_ref, oh_ref, ptab_ref, k_ref, gamma_ref, beta_ref, o_ref):
    dn = (((1,), (0,)), ((), ()))
    hi = lax.Precision.DEFAULT
    x = x_ref[...]
    segc = lax.dot_general(
        oh_ref[0], ptab_ref[...], (((0,), (0,)), ((), ())), precision=hi
    )
    e = x + pos_ref[...] + segc
    k = k_ref[...]
    m = lax.dot_general(e, k, dn, precision=hi)
    q = lax.dot_general(e * e, k, dn, precision=hi)
    y = (e - m) * lax.rsqrt(q - m * m + 1e-6)
    o_ref[...] = y * gamma_ref[...] + beta_ref[...]


def _tc_fuse2(rows2, pos_tiled, oh, ptab, kmat, gamma2, beta2):
    return pl.pallas_call(
        _tc_fuse2_body,
        grid=(_NP // _BP,),
        in_specs=[
            pl.BlockSpec((_BP, _D2), lambda i: (i, 0)),
            pl.BlockSpec((_BP, _D2), lambda i: (0, 0)),
            pl.BlockSpec((1, 16, _BP), lambda i: (i, 0, 0)),
            pl.BlockSpec((16, _D2), lambda i: (0, 0)),
            pl.BlockSpec((_D2, _D2), lambda i: (0, 0)),
            pl.BlockSpec((1, _D2), lambda i: (0, 0)),
            pl.BlockSpec((1, _D2), lambda i: (0, 0)),
        ],
        out_specs=pl.BlockSpec((_BP, _D2), lambda i: (i, 0)),
        out_shape=jax.ShapeDtypeStruct((_NP, _D2), jnp.float32),
    )(rows2, pos_tiled, oh, ptab, kmat, gamma2, beta2)


def _stream(src, seg, word_emb, pos_emb, seg_emb, gamma, beta):
    idx = src.reshape(-1).astype(jnp.int32)
    rows2 = _make_sc_gather()(idx, word_emb).reshape(_NP, _D2)
    pos_pairs = pos_emb[:_S].reshape(_S // 2, _D2)
    pos_tiled = jnp.tile(pos_pairs, (_BP // (_S // 2), 1))
    sp = seg.astype(jnp.int32).reshape(_NP, 2)
    code = sp[:, 0] * _NSEG + sp[:, 1]
    oh = jax.nn.one_hot(code, 16, dtype=jnp.float32, axis=0)
    oh = oh.reshape(16, _NP // _BP, _BP).transpose(1, 0, 2)
    ptab = jnp.zeros((16, _D2), jnp.float32).at[: _NSEG * _NSEG].set(
        jnp.concatenate(
            [jnp.repeat(seg_emb, _NSEG, axis=0), jnp.tile(seg_emb, (_NSEG, 1))], axis=1
        )
    )
    half = jnp.arange(_D2) // _D
    kmat = jnp.where(half[:, None] == half[None, :], 1.0 / _D, 0.0).astype(jnp.float32)
    gamma2 = jnp.tile(gamma, 2).reshape(1, _D2)
    beta2 = jnp.tile(beta, 2).reshape(1, _D2)
    out2 = _tc_fuse2(rows2, pos_tiled, oh, ptab, kmat, gamma2, beta2)
    return out2.reshape(_B, _S, _D)


def kernel(src_0, src_1, seg_0, seg_1,
           word_emb_0, pos_emb_0, seg_emb_0, gamma_0, beta_0,
           word_emb_1, pos_emb_1, seg_emb_1, gamma_1, beta_1):
    out0 = _stream(src_0, seg_0, word_emb_0, pos_emb_0, seg_emb_0, gamma_0, beta_0)
    out1 = _stream(src_1, seg_1, word_emb_1, pos_emb_1, seg_emb_1, gamma_1, beta_1)
    return (out0, out1)
